# baseline (speedup 1.0000x reference)
import jax
import jax.numpy as jnp
from jax.experimental import pallas as pl
from jax.experimental.pallas import tpu as pltpu

_B, _S, _H = 4, 4096, 4096
_E = 64
_K = 8
_G = 64
_T = 1024
_HC = _H // 2


def _gate_kernel(x0_ref, x1_ref, wt_ref, b_ref, idx_ref, w_ref, cnt_ref,
                 ssum_ref):
    wt = wt_ref[...]
    logits = (jnp.dot(x0_ref[...], wt[:_HC], preferred_element_type=jnp.float32)
              + jnp.dot(x1_ref[...], wt[_HC:], preferred_element_type=jnp.float32)
              + b_ref[...])

    ex = jnp.exp(logits)
    scores = ex / jnp.sum(ex, axis=-1, keepdims=True)

    iota = jax.lax.broadcasted_iota(jnp.int32, scores.shape, 1)
    rev = (jnp.float32(_E) - iota.astype(jnp.float32))
    vals = scores
    top_vals = []
    top_ridx = []
    for _ in range(_K):
        m = jnp.max(vals, axis=-1, keepdims=True)
        r = jnp.max(jnp.where(vals == m, rev, 0.0), axis=-1, keepdims=True)
        top_vals.append(m)
        top_ridx.append(r)
        vals = jnp.where(rev == r, -1.0, vals)

    tv = jnp.concatenate(top_vals, axis=1)
    tr = jnp.concatenate(top_ridx, axis=1)
    denom = jnp.sum(tv, axis=-1, keepdims=True) + 1e-20
    idx_ref[...] = (jnp.float32(_E) - tr).astype(jnp.int32)
    w_ref[...] = tv / denom

    selected = (vals < -0.5).astype(jnp.float32)
    cnt_ref[...] = jnp.sum(selected, axis=0).reshape(1, 1, _E)
    ssum_ref[...] = jnp.sum(scores, axis=0).reshape(1, 1, _E)


def _aux_kernel(cnt_ref, ssum_ref, aux_ref):
    c = jnp.sum(cnt_ref[...], axis=1)
    s = jnp.sum(ssum_ref[...], axis=1)
    scale = _G / (_S * _K * _S * _B)
    aux_ref[...] = (jnp.sum(c * s) * scale).reshape(1, 1)


@jax.jit
def _run(x, weight, bias):
    hidden = x.reshape(-1, _H)
    wt = weight.T
    b2 = bias.reshape(1, _E)
    n = hidden.shape[0]
    nb = n // _T
    topk_idx, topk_weight, cnt, ssum = pl.pallas_call(
        _gate_kernel,
        grid=(nb,),
        in_specs=[
            pl.BlockSpec((_T, _HC), lambda i: (i, 0)),
            pl.BlockSpec((_T, _HC), lambda i: (i, 1)),
            pl.BlockSpec((_H, _E), lambda i: (0, 0)),
            pl.BlockSpec((1, _E), lambda i: (0, 0)),
        ],
        out_specs=[
            pl.BlockSpec((_T, _K), lambda i: (i, 0)),
            pl.BlockSpec((_T, _K), lambda i: (i, 0)),
            pl.BlockSpec((1, 1, _E), lambda i: (i, 0, 0)),
            pl.BlockSpec((1, 1, _E), lambda i: (i, 0, 0)),
        ],
        out_shape=[
            jax.ShapeDtypeStruct((n, _K), jnp.int32),
            jax.ShapeDtypeStruct((n, _K), jnp.float32),
            jax.ShapeDtypeStruct((nb, 1, _E), jnp.float32),
            jax.ShapeDtypeStruct((nb, 1, _E), jnp.float32),
        ],
        compiler_params=pltpu.CompilerParams(
            dimension_semantics=("arbitrary",),
        ),
    )(hidden, hidden, wt, b2)

    bpb = nb // _B
    aux = pl.pallas_call(
        _aux_kernel,
        out_shape=jax.ShapeDtypeStruct((1, 1), jnp.float32),
    )(cnt.reshape(_B, bpb, _E), ssum.reshape(_B, bpb, _E))
    return topk_idx, topk_weight, aux[0, 0]


def kernel(x, weight, bias):
    return _run(x, weight, bias)


# speedup vs baseline: 1.0102x; 1.0102x over previous
import jax
import jax.numpy as jnp
from jax.experimental import pallas as pl
from jax.experimental.pallas import tpu as pltpu

_B, _S, _H = 4, 4096, 4096
_E = 64
_K = 8
_G = 64
_T = 1024
_HC = _H // 2


def _gate_kernel(x0_ref, x1_ref, wt_ref, b_ref, idx_ref, w_ref, aux_ref,
                 cnt_acc, ssum_acc):
    pid = pl.program_id(0)
    nsteps = pl.num_programs(0)

    @pl.when(pid == 0)
    def _init():
        cnt_acc[...] = jnp.zeros_like(cnt_acc)
        ssum_acc[...] = jnp.zeros_like(ssum_acc)

    wt = wt_ref[...]
    logits = (jnp.dot(x0_ref[...], wt[:_HC], preferred_element_type=jnp.float32)
              + jnp.dot(x1_ref[...], wt[_HC:], preferred_element_type=jnp.float32)
              + b_ref[...])

    ex = jnp.exp(logits)
    scores = ex / jnp.sum(ex, axis=-1, keepdims=True)

    iota = jax.lax.broadcasted_iota(jnp.int32, scores.shape, 1)
    rev = (jnp.float32(_E) - iota.astype(jnp.float32))
    vals = scores
    top_vals = []
    top_ridx = []
    for _ in range(_K):
        m = jnp.max(vals, axis=-1, keepdims=True)
        r = jnp.max(jnp.where(vals == m, rev, 0.0), axis=-1, keepdims=True)
        top_vals.append(m)
        top_ridx.append(r)
        vals = jnp.where(rev == r, -1.0, vals)

    tv = jnp.concatenate(top_vals, axis=1)
    tr = jnp.concatenate(top_ridx, axis=1)
    denom = jnp.sum(tv, axis=-1, keepdims=True) + 1e-20
    idx_ref[...] = (jnp.float32(_E) - tr).astype(jnp.int32)
    w_ref[...] = tv / denom

    blocks_per_batch = _S // _T
    b = pid // blocks_per_batch
    selected = (vals < -0.5).astype(jnp.float32)
    cnt = jnp.sum(selected, axis=0, keepdims=True)
    ssum = jnp.sum(scores, axis=0, keepdims=True)
    rows = jax.lax.broadcasted_iota(jnp.int32, (_B, _E), 0)
    hit = (rows == b).astype(jnp.float32)
    cnt_acc[...] += hit * cnt
    ssum_acc[...] += hit * ssum

    @pl.when(pid == nsteps - 1)
    def _finish():
        scale = _G / (_S * _K * _S * _B)
        aux_ref[...] = (jnp.sum(cnt_acc[...] * ssum_acc[...]) * scale).reshape(1, 1)


@jax.jit
def _run(x, weight, bias):
    hidden = x.reshape(-1, _H)
    wt = weight.T
    b2 = bias.reshape(1, _E)
    n = hidden.shape[0]
    nb = n // _T
    topk_idx, topk_weight, aux = pl.pallas_call(
        _gate_kernel,
        grid=(nb,),
        in_specs=[
            pl.BlockSpec((_T, _HC), lambda i: (i, 0)),
            pl.BlockSpec((_T, _HC), lambda i: (i, 1)),
            pl.BlockSpec((_H, _E), lambda i: (0, 0)),
            pl.BlockSpec((1, _E), lambda i: (0, 0)),
        ],
        out_specs=[
            pl.BlockSpec((_T, _K), lambda i: (i, 0)),
            pl.BlockSpec((_T, _K), lambda i: (i, 0)),
            pl.BlockSpec((1, 1), lambda i: (0, 0)),
        ],
        out_shape=[
            jax.ShapeDtypeStruct((n, _K), jnp.int32),
            jax.ShapeDtypeStruct((n, _K), jnp.float32),
            jax.ShapeDtypeStruct((1, 1), jnp.float32),
        ],
        scratch_shapes=[
            pltpu.VMEM((_B, _E), jnp.float32),
            pltpu.VMEM((_B, _E), jnp.float32),
        ],
        compiler_params=pltpu.CompilerParams(
            dimension_semantics=("arbitrary",),
        ),
    )(hidden, hidden, wt, b2)
    return topk_idx, topk_weight, aux[0, 0]


def kernel(x, weight, bias):
    return _run(x, weight, bias)
